# baseline (speedup 1.0000x reference)
import functools

import jax
import jax.numpy as jnp
from jax import lax
from jax.experimental import pallas as pl
from jax.experimental.pallas import tpu as pltpu
from jax.experimental.pallas import tpu_sc as plsc

EMBED = 128
NC = 2
NS = 16
NW = NC * NS
G = 64
NB = 10
IC = 8


def _sc_gather(xf2, table):
    rows_total, g = xf2.shape
    assert g == G
    B = rows_total * G
    per_w_rows = rows_total // NW
    nb_iter = per_w_rows // NB
    nchunk = per_w_rows // (NB * IC)
    assert per_w_rows % (NB * IC) == 0
    mesh = plsc.VectorSubcoreMesh(core_axis_name="c", subcore_axis_name="s")

    scratch = [
        pltpu.VMEM((2, IC * NB, G), jnp.int32),
        pltpu.VMEM((NB, G, EMBED), jnp.float32),
        pltpu.VMEM_SHARED((32, EMBED), jnp.float32),
        pltpu.SemaphoreType.DMA,
    ] + [pltpu.SemaphoreType.DMA] * (2 * NB)

    @functools.partial(
        pl.kernel,
        mesh=mesh,
        out_type=jax.ShapeDtypeStruct((B, EMBED), jnp.float32),
        scratch_types=scratch,
    )
    def k(idx_hbm, table_hbm, out_hbm, idx_v, rows, table_sp, i_sem, *sems):
        g_sems = sems[:NB]
        st_sems = sems[NB:]
        wid = lax.axis_index("s") * NC + lax.axis_index("c")
        base_row = wid * per_w_rows
        CH = IC * NB

        @pl.when(lax.axis_index("s") == 0)
        def _load_table():
            pltpu.sync_copy(table_hbm, table_sp)

        pltpu.async_copy(idx_hbm.at[pl.ds(base_row, CH)], idx_v.at[0], i_sem)
        plsc.subcore_barrier()

        def body(j, carry):
            row0 = base_row + j * NB
            t = lax.div(j, IC)
            par = lax.rem(t, 2)

            @pl.when(lax.rem(j, IC) == 0)
            def _stage():
                pltpu.make_async_copy(
                    idx_hbm.at[pl.ds(pl.multiple_of(base_row + t * CH, 8), CH)],
                    idx_v.at[par], i_sem).wait()

                @pl.when(t + 1 < nchunk)
                def _prefetch():
                    pltpu.async_copy(
                        idx_hbm.at[pl.ds(
                            pl.multiple_of(base_row + (t + 1) * CH, 8), CH)],
                        idx_v.at[1 - par], i_sem)

            ib = lax.rem(j, IC) * NB
            for b in range(NB):
                @pl.when(j > 0)
                def _drain(b=b):
                    pltpu.make_async_copy(
                        rows.at[b], out_hbm.at[pl.ds((row0 + b) * G, G)],
                        st_sems[b]).wait()
                pltpu.async_copy(table_sp.at[idx_v.at[par, ib + b]], rows.at[b],
                                 g_sems[b])
            for b in range(NB):
                pltpu.make_async_copy(table_sp.at[idx_v.at[par, ib + b]],
                                      rows.at[b], g_sems[b]).wait()
                pltpu.async_copy(rows.at[b], out_hbm.at[pl.ds((row0 + b) * G, G)],
                                 st_sems[b])
            return carry

        lax.fori_loop(0, nb_iter, body, 0)
        row_last = base_row + (nb_iter - 1) * NB
        for b in range(NB):
            pltpu.make_async_copy(
                rows.at[b], out_hbm.at[pl.ds((row_last + b) * G, G)],
                st_sems[b]).wait()

    return k(xf2, table)


TC_BLK = 2048


def _tc_onehot_body(x_ref, t_ref, o_ref):
    sub = TC_BLK // 128
    idx = x_ref[0]
    vocab = jax.lax.broadcasted_iota(jnp.int32, (sub, 128, 32), 2)
    onehot = (idx[:, :, None] == vocab).astype(jnp.float32)
    o_ref[0] = jax.lax.dot_general(
        onehot, t_ref[...], (((2,), (0,)), ((), ())),
        preferred_element_type=jnp.float32)


def _tc_onehot(xf, table):
    B = xf.shape[0]
    nblk = B // TC_BLK
    sub = TC_BLK // 128
    x3 = xf.reshape(nblk, sub, 128)
    out = pl.pallas_call(
        _tc_onehot_body,
        grid=(nblk,),
        in_specs=[
            pl.BlockSpec((1, sub, 128), lambda i: (i, 0, 0)),
            pl.BlockSpec((32, EMBED), lambda i: (0, 0)),
        ],
        out_specs=pl.BlockSpec((1, sub, 128, EMBED), lambda i: (i, 0, 0, 0)),
        out_shape=jax.ShapeDtypeStruct((nblk, sub, 128, EMBED), jnp.float32),
    )(x3, table)
    return out.reshape(B, EMBED)


B_SC_FRAC_UNITS = 13


def kernel(x, table):
    n, s = x.shape
    B = n * s
    unit = G * NW * NB * IC
    b_sc = B_SC_FRAC_UNITS * unit
    xf = x.reshape(B)
    out_sc = _sc_gather(xf[:b_sc].reshape(b_sc // G, G), table)
    out_tc = _tc_onehot(xf[b_sc:], table)
    out = jnp.concatenate([out_sc, out_tc], axis=0)
    return out.reshape(n, s, EMBED)
